# baseline (speedup 1.0000x reference)
import functools

import jax
import jax.numpy as jnp
from jax import lax
from jax.experimental import pallas as pl
from jax.experimental.pallas import tpu as pltpu
from jax.experimental.pallas import tpu_sc as plsc

_NUM_CORES = 2


@functools.lru_cache(maxsize=None)
def _make_broadcast(batch, seq_len, feat):
    rows_per_core = seq_len // _NUM_CORES
    mesh = plsc.ScalarSubcoreMesh(axis_name="c", num_cores=_NUM_CORES)

    @jax.jit
    @functools.partial(
        pl.kernel,
        mesh=mesh,
        out_type=jax.ShapeDtypeStruct((batch, seq_len, feat), jnp.float32),
        scratch_types=[
            pltpu.VMEM_SHARED((rows_per_core, feat), jnp.float32),
            pltpu.SemaphoreType.DMA,
        ],
    )
    def k(w_hbm, out_hbm, spmem, sem):
        cid = lax.axis_index("c")
        base = cid * rows_per_core
        pltpu.sync_copy(w_hbm.at[pl.ds(base, rows_per_core)], spmem)
        stores = [
            pltpu.async_copy(
                spmem, out_hbm.at[b, pl.ds(base, rows_per_core)], sem
            )
            for b in range(batch)
        ]
        for s in stores:
            s.wait()

    return k


def kernel(seq_in_embeds, W):
    batch, seq_len = seq_in_embeds.shape[0], seq_in_embeds.shape[1]
    return _make_broadcast(batch, seq_len, W.shape[1])(W)
